# Initial kernel scaffold; baseline (speedup 1.0000x reference)
import jax
import jax.numpy as jnp
from jax.experimental import pallas as pl


def kernel(adj, x, pseudo, W1, r1, b1, W2, r2, b2, W3, r3, b3):
    raise NotImplementedError("write your pallas kernel here")


# speedup vs baseline: 2.8185x; 2.8185x over previous
import functools

import jax
import jax.numpy as jnp
from jax import lax
from jax.experimental import pallas as pl
from jax.experimental.pallas import tpu as pltpu
from jax.experimental.pallas import tpu_sc as plsc

N = 10000
E = 320000
K = 48
NC, NS, LANES = 2, 16, 16
NW = NC * NS
BB = 32
NB_TILE = 320
E_PAD = NW * BB * NB_TILE
R_ROWS = E_PAD // BB
CH = 64
NCH = NB_TILE // CH
F3R = NB_TILE * BB // 128
N_ACC = 10112
QTR = 2528
ACCQ = 2560
GB = 4


def _bcast(vec16, lane):
    idx = jnp.full((LANES, 1), lane, jnp.int32)
    return lax.gather(
        vec16, idx,
        dimension_numbers=lax.GatherDimensionNumbers(
            offset_dims=(), collapsed_slice_dims=(0,), start_index_map=(0,)),
        slice_sizes=(1,), mode=lax.GatherScatterMode.PROMISE_IN_BOUNDS)


def _pre_body(src_ref, p1_ref, p2_ref, msk_ref, gidx_ref, wq_ref):
    src = src_ref[...]
    v1 = p1_ref[...] * 2.0
    lo1 = jnp.clip(jnp.floor(v1), 0.0, 1.0)
    f1 = v1 - lo1
    v2 = p2_ref[...] * 7.0
    lo2 = jnp.clip(jnp.floor(v2), 0.0, 6.0)
    f2 = v2 - lo2
    base = src * 24 + lo1.astype(jnp.int32) * 8 + lo2.astype(jnp.int32)
    m = msk_ref[...]
    g1 = 1.0 - f1
    g2 = 1.0 - f2
    gidx_ref[...] = jnp.concatenate([base, base + 1, base + 8, base + 9], axis=1)
    wq_ref[...] = jnp.concatenate(
        [g1 * g2 * m, g1 * f2 * m, f1 * g2 * m, f1 * f2 * m], axis=1)


def _precompute(srcp, p1, p2, msk):
    nblk = R_ROWS // 32
    return pl.pallas_call(
        _pre_body,
        grid=(nblk,),
        in_specs=[pl.BlockSpec((32, 32), lambda i: (i, 0))] * 4,
        out_specs=[pl.BlockSpec((32, 128), lambda i: (i, 0))] * 2,
        out_shape=[
            jax.ShapeDtypeStruct((R_ROWS, 128), jnp.int32),
            jax.ShapeDtypeStruct((R_ROWS, 128), jnp.float32),
        ],
    )(srcp, p1, p2, msk)


def _gather_body(nh, out_g, deg_col, z2_hbm, gidx_hbm, wq_hbm, f3_hbm,
                 m_hbm, gidxc, wqc, f3blk, rows, mbuf, gsem):
    c = lax.axis_index("c")
    s = lax.axis_index("s")
    wid = c * NS + s
    row0 = wid * NB_TILE
    onehot0 = 1.0 - jnp.minimum(
        lax.iota(jnp.int32, LANES).astype(jnp.float32), 1.0)
    pltpu.sync_copy(f3_hbm.at[pl.ds(wid * F3R, F3R)], f3blk)

    def zrow(i, carry):
        for t in range(8):
            mbuf[i, pl.ds(t * LANES, LANES)] = jnp.zeros((LANES,),
                                                         jnp.float32)
        return carry

    lax.fori_loop(0, BB, zrow, 0)

    for ch in range(NCH):
        base_r = row0 + ch * CH
        pltpu.sync_copy(gidx_hbm.at[pl.ds(base_r, CH)], gidxc)
        pltpu.sync_copy(wq_hbm.at[pl.ds(base_r, CH)], wqc)

        def batch_body(i, carry):
            pltpu.async_copy(z2_hbm.at[gidxc.at[i]], rows, gsem).wait()
            bi = ch * CH + i

            def edge_body(j, carry2):
                off = bi * BB + j
                jhi = (j // LANES) * LANES
                jlo = j % LANES
                f3v = f3blk[off // 128, pl.ds(((off % 128) // LANES) * LANES,
                                              LANES)]
                f3 = _bcast(f3v, (off % 128) % LANES)
                omf3 = 1.0 - f3
                w0 = _bcast(wqc[i, pl.ds(jhi, LANES)], jlo)
                w1 = _bcast(wqc[i, pl.ds(32 + jhi, LANES)], jlo)
                w2v = _bcast(wqc[i, pl.ds(64 + jhi, LANES)], jlo)
                w3 = _bcast(wqc[i, pl.ds(96 + jhi, LANES)], jlo)
                for h in range(nh):
                    dlo = pl.ds(h * LANES, LANES)
                    dhi = pl.ds(out_g + h * LANES, LANES)
                    alo = (rows[j, dlo] * w0 + rows[32 + j, dlo] * w1
                           + rows[64 + j, dlo] * w2v + rows[96 + j, dlo] * w3)
                    ahi = (rows[j, dhi] * w0 + rows[32 + j, dhi] * w1
                           + rows[64 + j, dhi] * w2v + rows[96 + j, dhi] * w3)
                    mbuf[j, pl.ds(h * LANES, LANES)] = alo * omf3 + ahi * f3
                if deg_col:
                    mbuf[j, pl.ds(nh * LANES, LANES)] = (
                        (w0 + w1 + w2v + w3) * onehot0)
                return carry2

            lax.fori_loop(0, BB, edge_body, 0)
            pltpu.sync_copy(mbuf, m_hbm.at[pl.ds((row0 + bi) * BB, BB)])
            return carry

        lax.fori_loop(0, CH, batch_body, 0)


def _sc_gather(nh, out_g, deg_col, z2, gidx, wq, f3):
    mesh = plsc.VectorSubcoreMesh(core_axis_name="c", subcore_axis_name="s")
    return pl.kernel(
        functools.partial(_gather_body, nh, out_g, deg_col),
        out_type=jax.ShapeDtypeStruct((E_PAD, 128), jnp.float32),
        mesh=mesh,
        scratch_types=[
            pltpu.VMEM((CH, 128), jnp.int32),
            pltpu.VMEM((CH, 128), jnp.float32),
            pltpu.VMEM((F3R, 128), jnp.float32),
            pltpu.VMEM((4 * BB, 2 * out_g), jnp.float32),
            pltpu.VMEM((BB, 128), jnp.float32),
            pltpu.SemaphoreType.DMA,
        ],
    )(z2, gidx, wq, f3)


def _scatter_body(k, m_hbm, dst_hbm, part_hbm, dstblk, idxbuf, mrows, obuf,
                  obidx, acc_sh):
    c = lax.axis_index("c")
    s = lax.axis_index("s")
    nrow = R_ROWS // NS
    lo = (2 * k + c) * QTR
    rps = ACCQ // NS

    def zrow(i, carry):
        for t in range(8):
            mrows[i, pl.ds(t * LANES, LANES)] = jnp.zeros((LANES,),
                                                          jnp.float32)
            obuf[i % 32, pl.ds(t * LANES, LANES)] = jnp.zeros((LANES,),
                                                             jnp.float32)
        return carry

    lax.fori_loop(0, GB * BB, zrow, 0)

    def fill_idx(base_row):
        for u in range(8):
            idxbuf[0, pl.ds(u * LANES, LANES)] = (
                lax.iota(jnp.int32, LANES) + (base_row + u * LANES))

    fill_idx(s * rps)
    pltpu.sync_copy(mrows, acc_sh.at[idxbuf.at[0]])
    for u in range(2):
        obidx[0, pl.ds(u * LANES, LANES)] = (
            lax.iota(jnp.int32, LANES) + (s * rps + 128 + u * LANES))
    pltpu.sync_copy(obuf, acc_sh.at[obidx.at[0]])

    pltpu.sync_copy(dst_hbm.at[pl.ds(s * nrow, nrow)], dstblk)
    plsc.subcore_barrier()

    def body(i, carry):
        r = s * nrow + i * GB
        pltpu.sync_copy(m_hbm.at[pl.ds(r * BB, GB * BB)], mrows)
        for u in range(GB * BB // LANES):
            d = dstblk[i * GB + u // 2, pl.ds((u % 2) * LANES, LANES)]
            local = d - lo
            bad = (local >> 31) | ((QTR - 1 - local) >> 31)
            idxbuf[0, pl.ds(u * LANES, LANES)] = (
                (local & ~bad) | (QTR & bad))
        pltpu.sync_copy(mrows, acc_sh.at[idxbuf.at[0]], add=True)
        return carry

    lax.fori_loop(0, nrow // GB, body, 0)
    plsc.subcore_barrier()

    fill_idx(s * rps)
    pltpu.sync_copy(acc_sh.at[idxbuf.at[0]], mrows)
    pltpu.sync_copy(mrows, part_hbm.at[c, pl.ds(s * rps, GB * BB)])
    pltpu.sync_copy(acc_sh.at[obidx.at[0]], obuf)
    pltpu.sync_copy(obuf, part_hbm.at[c, pl.ds(s * rps + 128, 32)])


def _sc_scatter(k, m, dstp):
    mesh = plsc.VectorSubcoreMesh(core_axis_name="c", subcore_axis_name="s")
    return pl.kernel(
        functools.partial(_scatter_body, k),
        out_type=jax.ShapeDtypeStruct((NC, ACCQ, 128), jnp.float32),
        mesh=mesh,
        scratch_types=[
            pltpu.VMEM((R_ROWS // NS, BB), jnp.int32),
            pltpu.VMEM((1, GB * BB), jnp.int32),
            pltpu.VMEM((GB * BB, 128), jnp.float32),
            pltpu.VMEM((32, 128), jnp.float32),
            pltpu.VMEM((1, 32), jnp.int32),
            pltpu.VMEM_SHARED((ACCQ, 128), jnp.float32),
        ],
    )(m, dstp)


def _sc_layer(nh, out_g, deg_col, z2, gidx, wq, dstp, f3):
    m = _sc_gather(nh, out_g, deg_col, z2, gidx, wq, f3)
    p0 = _sc_scatter(0, m, dstp)
    m2 = lax.optimization_barrier((m, p0))[0]
    p1 = _sc_scatter(1, m2, dstp)
    acc = jnp.concatenate(
        [p0[0, :QTR], p0[1, :QTR], p1[0, :QTR], p1[1, :QTR]])[:N]
    return acc


def _mm_body(h_ref, w_ref, o_ref):
    o_ref[...] = jnp.dot(h_ref[...], w_ref[...],
                         preferred_element_type=jnp.float32)


def _z_matmul(h, wr):
    in_c = h.shape[1]
    kout = wr.shape[1]
    nb = 1000
    ob = min(kout, 1536)
    return pl.pallas_call(
        _mm_body,
        grid=(N // nb, kout // ob),
        in_specs=[
            pl.BlockSpec((nb, in_c), lambda i, j: (i, 0)),
            pl.BlockSpec((in_c, ob), lambda i, j: (0, j)),
        ],
        out_specs=pl.BlockSpec((nb, ob), lambda i, j: (i, j)),
        out_shape=jax.ShapeDtypeStruct((N, kout), jnp.float32),
    )(h, wr)


def _combine_body(final, part_ref, deg_ref, h_ref, r_ref, b_ref, o_ref):
    deg = deg_ref[...]
    out = part_ref[...] / jnp.maximum(deg, 1.0) + jnp.dot(
        h_ref[...], r_ref[...], preferred_element_type=jnp.float32) + b_ref[...]
    out = jnp.maximum(out, 0.0)
    if final:
        mx = jnp.max(out, axis=1, keepdims=True)
        out = out - mx - jnp.log(
            jnp.sum(jnp.exp(out - mx), axis=1, keepdims=True))
    o_ref[...] = out


def _combine(part, deg, h, r, b, final=False):
    out_c = r.shape[1]
    in_c = h.shape[1]
    nb = 1000
    return pl.pallas_call(
        functools.partial(_combine_body, final),
        grid=(N // nb,),
        in_specs=[
            pl.BlockSpec((nb, out_c), lambda i: (i, 0)),
            pl.BlockSpec((nb, 1), lambda i: (i, 0)),
            pl.BlockSpec((nb, in_c), lambda i: (i, 0)),
            pl.BlockSpec((in_c, out_c), lambda i: (0, 0)),
            pl.BlockSpec((1, out_c), lambda i: (0, 0)),
        ],
        out_specs=pl.BlockSpec((nb, out_c), lambda i: (i, 0)),
        out_shape=jax.ShapeDtypeStruct((N, out_c), jnp.float32),
    )(part, deg, h, r, b)


def kernel(adj, x, pseudo, W1, r1, b1, W2, r2, b2, W3, r3, b3):
    pad = E_PAD - E
    src = adj[0].astype(jnp.int32)
    dst = adj[1].astype(jnp.int32)
    srcp = jnp.pad(src, (0, pad)).reshape(R_ROWS, BB)
    dstp = jnp.pad(dst, (0, pad)).reshape(R_ROWS, BB)
    p1 = jnp.pad(pseudo[:, 0], (0, pad)).reshape(R_ROWS, BB)
    p2 = jnp.pad(pseudo[:, 1], (0, pad)).reshape(R_ROWS, BB)
    f3 = jnp.pad(pseudo[:, 2], (0, pad)).reshape(E_PAD // 128, 128)
    msk = jnp.pad(jnp.ones((E,), jnp.float32), (0, pad)).reshape(R_ROWS, BB)

    gidx, wq = _precompute(srcp, p1, p2, msk)

    W1p = jnp.pad(W1, ((0, 0), (0, 0), (0, 32)))
    wr1 = W1p.transpose(1, 0, 2).reshape(1, K * 64)
    z2 = _z_matmul(x, wr1).reshape(N * 24, 128)
    acc1 = _sc_layer(2, 64, True, z2, gidx, wq, dstp, f3)
    deg = acc1[:, 32:33]
    h = _combine(acc1[:, :32], deg, x, r1, b1.reshape(1, 32), final=False)

    wr2 = W2.transpose(1, 0, 2).reshape(32, K * 64)
    z2 = _z_matmul(h, wr2).reshape(N * 24, 128)
    acc2 = _sc_layer(4, 64, False, z2, gidx, wq, dstp, f3)
    h = _combine(acc2[:, :64], deg, h, r2, b2.reshape(1, 64), final=False)

    wr3 = W3.transpose(1, 0, 2).reshape(64, K * 128)
    z2 = _z_matmul(h, wr3).reshape(N * 24, 256)
    acc3 = _sc_layer(8, 128, False, z2, gidx, wq, dstp, f3)
    h = _combine(acc3, deg, h, r3, b3.reshape(1, 128), final=True)
    return h


# speedup vs baseline: 2.8400x; 1.0076x over previous
import functools

import jax
import jax.numpy as jnp
from jax import lax
from jax.experimental import pallas as pl
from jax.experimental.pallas import tpu as pltpu
from jax.experimental.pallas import tpu_sc as plsc

N = 10000
E = 320000
K = 48
NC, NS, LANES = 2, 16, 16
NW = NC * NS
BB = 32
NB_TILE = 320
E_PAD = NW * BB * NB_TILE
R_ROWS = E_PAD // BB
CH = 64
NCH = NB_TILE // CH
F3R = NB_TILE * BB // 128
N_ACC = 10112
QTR = 2528
ACCQ = 2560
GB = 4


def _bcast(vec16, lane):
    idx = jnp.full((LANES, 1), lane, jnp.int32)
    return lax.gather(
        vec16, idx,
        dimension_numbers=lax.GatherDimensionNumbers(
            offset_dims=(), collapsed_slice_dims=(0,), start_index_map=(0,)),
        slice_sizes=(1,), mode=lax.GatherScatterMode.PROMISE_IN_BOUNDS)


def _pre_body(src_ref, p1_ref, p2_ref, msk_ref, gidx_ref, wq_ref):
    src = src_ref[...]
    v1 = p1_ref[...] * 2.0
    lo1 = jnp.clip(jnp.floor(v1), 0.0, 1.0)
    f1 = v1 - lo1
    v2 = p2_ref[...] * 7.0
    lo2 = jnp.clip(jnp.floor(v2), 0.0, 6.0)
    f2 = v2 - lo2
    base = src * 24 + lo1.astype(jnp.int32) * 8 + lo2.astype(jnp.int32)
    m = msk_ref[...]
    g1 = 1.0 - f1
    g2 = 1.0 - f2
    gidx_ref[...] = jnp.concatenate([base, base + 1, base + 8, base + 9], axis=1)
    wq_ref[...] = jnp.concatenate(
        [g1 * g2 * m, g1 * f2 * m, f1 * g2 * m, f1 * f2 * m], axis=1)


def _precompute(srcp, p1, p2, msk):
    nblk = R_ROWS // 32
    return pl.pallas_call(
        _pre_body,
        grid=(nblk,),
        in_specs=[pl.BlockSpec((32, 32), lambda i: (i, 0))] * 4,
        out_specs=[pl.BlockSpec((32, 128), lambda i: (i, 0))] * 2,
        out_shape=[
            jax.ShapeDtypeStruct((R_ROWS, 128), jnp.int32),
            jax.ShapeDtypeStruct((R_ROWS, 128), jnp.float32),
        ],
    )(srcp, p1, p2, msk)


def _gather_body(nh, out_g, deg_col, z2_hbm, gidx_hbm, wq_hbm, f3_hbm,
                 m_hbm, gidxc, wqc, f3blk, rows, rows2, mbuf, gsem, gsem2):
    c = lax.axis_index("c")
    s = lax.axis_index("s")
    wid = c * NS + s
    row0 = wid * NB_TILE
    onehot0 = 1.0 - jnp.minimum(
        lax.iota(jnp.int32, LANES).astype(jnp.float32), 1.0)
    pltpu.sync_copy(f3_hbm.at[pl.ds(wid * F3R, F3R)], f3blk)

    def zrow(i, carry):
        for t in range(8):
            mbuf[i, pl.ds(t * LANES, LANES)] = jnp.zeros((LANES,),
                                                         jnp.float32)
        return carry

    lax.fori_loop(0, BB, zrow, 0)

    def compute_batch(bi, rows):
        def edge_body(j, carry2):
            off = bi * BB + j
            jhi = (j // LANES) * LANES
            jlo = j % LANES
            f3v = f3blk[off // 128, pl.ds(((off % 128) // LANES) * LANES,
                                          LANES)]
            f3 = _bcast(f3v, (off % 128) % LANES)
            omf3 = 1.0 - f3
            i = (bi % CH)
            w0 = _bcast(wqc[i, pl.ds(jhi, LANES)], jlo)
            w1 = _bcast(wqc[i, pl.ds(32 + jhi, LANES)], jlo)
            w2v = _bcast(wqc[i, pl.ds(64 + jhi, LANES)], jlo)
            w3 = _bcast(wqc[i, pl.ds(96 + jhi, LANES)], jlo)
            for h in range(nh):
                dlo = pl.ds(h * LANES, LANES)
                dhi = pl.ds(out_g + h * LANES, LANES)
                alo = (rows[j, dlo] * w0 + rows[32 + j, dlo] * w1
                       + rows[64 + j, dlo] * w2v + rows[96 + j, dlo] * w3)
                ahi = (rows[j, dhi] * w0 + rows[32 + j, dhi] * w1
                       + rows[64 + j, dhi] * w2v + rows[96 + j, dhi] * w3)
                mbuf[j, pl.ds(h * LANES, LANES)] = alo * omf3 + ahi * f3
            if deg_col:
                mbuf[j, pl.ds(nh * LANES, LANES)] = (
                    (w0 + w1 + w2v + w3) * onehot0)
            return carry2

        lax.fori_loop(0, BB, edge_body, 0)
        pltpu.sync_copy(mbuf, m_hbm.at[pl.ds((row0 + bi) * BB, BB)])

    for ch in range(NCH):
        base_r = row0 + ch * CH
        pltpu.sync_copy(gidx_hbm.at[pl.ds(base_r, CH)], gidxc)
        pltpu.sync_copy(wq_hbm.at[pl.ds(base_r, CH)], wqc)

        def pair_body(p, carry):
            i0 = 2 * p
            i1 = i0 + 1
            cp0 = pltpu.async_copy(z2_hbm.at[gidxc.at[i0]], rows, gsem)
            cp1 = pltpu.async_copy(z2_hbm.at[gidxc.at[i1]], rows2, gsem2)
            cp0.wait()
            compute_batch(ch * CH + i0, rows)
            cp1.wait()
            compute_batch(ch * CH + i1, rows2)
            return carry

        lax.fori_loop(0, CH // 2, pair_body, 0)


def _sc_gather(nh, out_g, deg_col, z2, gidx, wq, f3):
    mesh = plsc.VectorSubcoreMesh(core_axis_name="c", subcore_axis_name="s")
    return pl.kernel(
        functools.partial(_gather_body, nh, out_g, deg_col),
        out_type=jax.ShapeDtypeStruct((E_PAD, 128), jnp.float32),
        mesh=mesh,
        scratch_types=[
            pltpu.VMEM((CH, 128), jnp.int32),
            pltpu.VMEM((CH, 128), jnp.float32),
            pltpu.VMEM((F3R, 128), jnp.float32),
            pltpu.VMEM((4 * BB, 2 * out_g), jnp.float32),
            pltpu.VMEM((4 * BB, 2 * out_g), jnp.float32),
            pltpu.VMEM((BB, 128), jnp.float32),
            pltpu.SemaphoreType.DMA,
            pltpu.SemaphoreType.DMA,
        ],
    )(z2, gidx, wq, f3)


def _scatter_body(k, m_hbm, dst_hbm, part_hbm, dstblk, idxbuf, mrows, obuf,
                  obidx, acc_sh):
    c = lax.axis_index("c")
    s = lax.axis_index("s")
    nrow = R_ROWS // NS
    lo = (2 * k + c) * QTR
    rps = ACCQ // NS

    def zrow(i, carry):
        for t in range(8):
            mrows[i, pl.ds(t * LANES, LANES)] = jnp.zeros((LANES,),
                                                          jnp.float32)
            obuf[i % 32, pl.ds(t * LANES, LANES)] = jnp.zeros((LANES,),
                                                             jnp.float32)
        return carry

    lax.fori_loop(0, GB * BB, zrow, 0)

    def fill_idx(base_row):
        for u in range(8):
            idxbuf[0, pl.ds(u * LANES, LANES)] = (
                lax.iota(jnp.int32, LANES) + (base_row + u * LANES))

    fill_idx(s * rps)
    pltpu.sync_copy(mrows, acc_sh.at[idxbuf.at[0]])
    for u in range(2):
        obidx[0, pl.ds(u * LANES, LANES)] = (
            lax.iota(jnp.int32, LANES) + (s * rps + 128 + u * LANES))
    pltpu.sync_copy(obuf, acc_sh.at[obidx.at[0]])

    pltpu.sync_copy(dst_hbm.at[pl.ds(s * nrow, nrow)], dstblk)
    plsc.subcore_barrier()

    def body(i, carry):
        r = s * nrow + i * GB
        pltpu.sync_copy(m_hbm.at[pl.ds(r * BB, GB * BB)], mrows)
        for u in range(GB * BB // LANES):
            d = dstblk[i * GB + u // 2, pl.ds((u % 2) * LANES, LANES)]
            local = d - lo
            bad = (local >> 31) | ((QTR - 1 - local) >> 31)
            idxbuf[0, pl.ds(u * LANES, LANES)] = (
                (local & ~bad) | (QTR & bad))
        pltpu.sync_copy(mrows, acc_sh.at[idxbuf.at[0]], add=True)
        return carry

    lax.fori_loop(0, nrow // GB, body, 0)
    plsc.subcore_barrier()

    fill_idx(s * rps)
    pltpu.sync_copy(acc_sh.at[idxbuf.at[0]], mrows)
    pltpu.sync_copy(mrows, part_hbm.at[c, pl.ds(s * rps, GB * BB)])
    pltpu.sync_copy(acc_sh.at[obidx.at[0]], obuf)
    pltpu.sync_copy(obuf, part_hbm.at[c, pl.ds(s * rps + 128, 32)])


def _sc_scatter(k, m, dstp):
    mesh = plsc.VectorSubcoreMesh(core_axis_name="c", subcore_axis_name="s")
    return pl.kernel(
        functools.partial(_scatter_body, k),
        out_type=jax.ShapeDtypeStruct((NC, ACCQ, 128), jnp.float32),
        mesh=mesh,
        scratch_types=[
            pltpu.VMEM((R_ROWS // NS, BB), jnp.int32),
            pltpu.VMEM((1, GB * BB), jnp.int32),
            pltpu.VMEM((GB * BB, 128), jnp.float32),
            pltpu.VMEM((32, 128), jnp.float32),
            pltpu.VMEM((1, 32), jnp.int32),
            pltpu.VMEM_SHARED((ACCQ, 128), jnp.float32),
        ],
    )(m, dstp)


def _sc_layer(nh, out_g, deg_col, z2, gidx, wq, dstp, f3):
    m = _sc_gather(nh, out_g, deg_col, z2, gidx, wq, f3)
    p0 = _sc_scatter(0, m, dstp)
    m2 = lax.optimization_barrier((m, p0))[0]
    p1 = _sc_scatter(1, m2, dstp)
    acc = jnp.concatenate(
        [p0[0, :QTR], p0[1, :QTR], p1[0, :QTR], p1[1, :QTR]])[:N]
    return acc


def _mm_body(h_ref, w_ref, o_ref):
    o_ref[...] = jnp.dot(h_ref[...], w_ref[...],
                         preferred_element_type=jnp.float32)


def _z_matmul(h, wr):
    in_c = h.shape[1]
    kout = wr.shape[1]
    nb = 1000
    ob = min(kout, 1536)
    return pl.pallas_call(
        _mm_body,
        grid=(N // nb, kout // ob),
        in_specs=[
            pl.BlockSpec((nb, in_c), lambda i, j: (i, 0)),
            pl.BlockSpec((in_c, ob), lambda i, j: (0, j)),
        ],
        out_specs=pl.BlockSpec((nb, ob), lambda i, j: (i, j)),
        out_shape=jax.ShapeDtypeStruct((N, kout), jnp.float32),
    )(h, wr)


def _combine_body(final, part_ref, deg_ref, h_ref, r_ref, b_ref, o_ref):
    deg = deg_ref[...]
    out = part_ref[...] / jnp.maximum(deg, 1.0) + jnp.dot(
        h_ref[...], r_ref[...], preferred_element_type=jnp.float32) + b_ref[...]
    out = jnp.maximum(out, 0.0)
    if final:
        mx = jnp.max(out, axis=1, keepdims=True)
        out = out - mx - jnp.log(
            jnp.sum(jnp.exp(out - mx), axis=1, keepdims=True))
    o_ref[...] = out


def _combine(part, deg, h, r, b, final=False):
    out_c = r.shape[1]
    in_c = h.shape[1]
    nb = 1000
    return pl.pallas_call(
        functools.partial(_combine_body, final),
        grid=(N // nb,),
        in_specs=[
            pl.BlockSpec((nb, out_c), lambda i: (i, 0)),
            pl.BlockSpec((nb, 1), lambda i: (i, 0)),
            pl.BlockSpec((nb, in_c), lambda i: (i, 0)),
            pl.BlockSpec((in_c, out_c), lambda i: (0, 0)),
            pl.BlockSpec((1, out_c), lambda i: (0, 0)),
        ],
        out_specs=pl.BlockSpec((nb, out_c), lambda i: (i, 0)),
        out_shape=jax.ShapeDtypeStruct((N, out_c), jnp.float32),
    )(part, deg, h, r, b)


def kernel(adj, x, pseudo, W1, r1, b1, W2, r2, b2, W3, r3, b3):
    pad = E_PAD - E
    src = adj[0].astype(jnp.int32)
    dst = adj[1].astype(jnp.int32)
    srcp = jnp.pad(src, (0, pad)).reshape(R_ROWS, BB)
    dstp = jnp.pad(dst, (0, pad)).reshape(R_ROWS, BB)
    p1 = jnp.pad(pseudo[:, 0], (0, pad)).reshape(R_ROWS, BB)
    p2 = jnp.pad(pseudo[:, 1], (0, pad)).reshape(R_ROWS, BB)
    f3 = jnp.pad(pseudo[:, 2], (0, pad)).reshape(E_PAD // 128, 128)
    msk = jnp.pad(jnp.ones((E,), jnp.float32), (0, pad)).reshape(R_ROWS, BB)

    gidx, wq = _precompute(srcp, p1, p2, msk)

    W1p = jnp.pad(W1, ((0, 0), (0, 0), (0, 32)))
    wr1 = W1p.transpose(1, 0, 2).reshape(1, K * 64)
    z2 = _z_matmul(x, wr1).reshape(N * 24, 128)
    acc1 = _sc_layer(2, 64, True, z2, gidx, wq, dstp, f3)
    deg = acc1[:, 32:33]
    h = _combine(acc1[:, :32], deg, x, r1, b1.reshape(1, 32), final=False)

    wr2 = W2.transpose(1, 0, 2).reshape(32, K * 64)
    z2 = _z_matmul(h, wr2).reshape(N * 24, 128)
    acc2 = _sc_layer(4, 64, False, z2, gidx, wq, dstp, f3)
    h = _combine(acc2[:, :64], deg, h, r2, b2.reshape(1, 64), final=False)

    wr3 = W3.transpose(1, 0, 2).reshape(64, K * 128)
    z2 = _z_matmul(h, wr3).reshape(N * 24, 256)
    acc3 = _sc_layer(8, 128, False, z2, gidx, wq, dstp, f3)
    h = _combine(acc3, deg, h, r3, b3.reshape(1, 128), final=True)
    return h
